# baseline (speedup 1.0000x reference)
import jax
import jax.numpy as jnp
from jax import lax
from jax.experimental import pallas as pl
from jax.experimental.pallas import tpu as pltpu
from jax.experimental.pallas import tpu_sc as plsc

_N = 10000
_E = 320000
_D = 128

_NC = 2
_NS = 16
_NW = _NC * _NS

_EPW = _E // _NW
_CH = 128
_CPB = 16
_NBLK = 5
_NCHUNK = _CPB * _NBLK
_EPW_PAD = _NCHUNK * _CH
_SUB = 32

_AGG_ROWS = 10112
_ZROWS = _AGG_ROWS // _NS
_TRASH = _N
_DEG_LEN = 10112


def _mm_t(a, b):
    return lax.dot_general(a, b, (((1,), (1,)), ((), ())),
                           preferred_element_type=jnp.float32,
                           precision=lax.Precision.HIGHEST)


def _msg_body(x_ref, w_ref, b_ref, o_ref):
    o_ref[...] = jnp.maximum(_mm_t(x_ref[...], w_ref[...]) + b_ref[...], 0.0)


def _compute_messages(x, w_c, b_c):
    blk = 1000
    grid = _N // blk
    return pl.pallas_call(
        _msg_body,
        grid=(grid,),
        in_specs=[
            pl.BlockSpec((blk, _D), lambda i: (i, 0)),
            pl.BlockSpec((_D, _D), lambda i: (0, 0)),
            pl.BlockSpec((1, _D), lambda i: (0, 0)),
        ],
        out_specs=pl.BlockSpec((blk, _D), lambda i: (i, 0)),
        out_shape=jax.ShapeDtypeStruct((_N, _D), jnp.float32),
    )(x, w_c, b_c.reshape(1, _D))


def _sc_body(m_hbm, src_hbm, dst_hbm,
             agg_out, deg_out,
             src_v, dst_v, rows_v, deg_v,
             agg_sh,
             sem0, sem1):
    c = lax.axis_index("c")
    s = lax.axis_index("s")
    w = c * _NS + s

    def _zrow(i, _):
        rows_v[0, i // 8, pl.ds((i % 8) * 16, 16)] = jnp.zeros((16,), jnp.float32)
        return 0
    lax.fori_loop(0, _CH * 8, _zrow, 0)
    for k in range(4):
        pltpu.sync_copy(rows_v.at[0], agg_sh.at[pl.ds(s * _ZROWS + k * _CH, _CH)])
    pltpu.sync_copy(rows_v.at[0].at[pl.ds(0, _ZROWS - 4 * _CH)],
                    agg_sh.at[pl.ds(s * _ZROWS + 4 * _CH, _ZROWS - 4 * _CH)])

    def _zdeg(i, _):
        deg_v[pl.ds(i * 16, 16)] = jnp.zeros((16,), jnp.float32)
        return 0
    lax.fori_loop(0, _DEG_LEN // 16, _zdeg, 0)

    plsc.subcore_barrier()

    ones16 = jnp.ones((16,), jnp.float32)

    def _fire(l, buf, sem):
        for h in range(_CH // _SUB):
            pltpu.async_copy(
                m_hbm.at[src_v.at[l].at[pl.ds(h * _SUB, _SUB)]],
                rows_v.at[buf].at[pl.ds(h * _SUB, _SUB)], sem)

    for b in range(_NBLK):
        pltpu.sync_copy(src_hbm.at[w].at[b], src_v)
        pltpu.sync_copy(dst_hbm.at[w].at[b], dst_v)

        _fire(0, 0, sem0)
        _fire(1, 1, sem1)

        def _deg_chunk(j, _):
            for q in range(_CH // 16):
                idx = dst_v[j, pl.ds(q * 16, 16)]
                plsc.addupdate_scatter(deg_v, [idx], ones16)
            return 0
        lax.fori_loop(0, _CPB, _deg_chunk, 0)

        def _pair(i, _):
            l0 = 2 * i
            pltpu.make_async_copy(m_hbm.at[src_v.at[l0]], rows_v.at[0], sem0).wait()
            pltpu.sync_copy(rows_v.at[0], agg_sh.at[dst_v.at[l0]], add=True)
            _fire(l0 + 2, 0, sem0)

            l1 = l0 + 1
            pltpu.make_async_copy(m_hbm.at[src_v.at[l1]], rows_v.at[1], sem1).wait()
            pltpu.sync_copy(rows_v.at[1], agg_sh.at[dst_v.at[l1]], add=True)
            _fire(l1 + 2, 1, sem1)
            return 0

        lax.fori_loop(0, _CPB // 2 - 1, _pair, 0)

        lt = _CPB - 2
        pltpu.make_async_copy(m_hbm.at[src_v.at[lt]], rows_v.at[0], sem0).wait()
        pltpu.sync_copy(rows_v.at[0], agg_sh.at[dst_v.at[lt]], add=True)
        pltpu.make_async_copy(m_hbm.at[src_v.at[lt + 1]], rows_v.at[1], sem1).wait()
        pltpu.sync_copy(rows_v.at[1], agg_sh.at[dst_v.at[lt + 1]], add=True)

    plsc.subcore_barrier()
    pltpu.sync_copy(agg_sh.at[pl.ds(s * _ZROWS, _ZROWS)],
                    agg_out.at[c, pl.ds(s * _ZROWS, _ZROWS)])
    pltpu.sync_copy(deg_v, deg_out.at[w])


def _sc_reduce(m, src_p, dst_p):
    mesh = plsc.VectorSubcoreMesh(core_axis_name="c", subcore_axis_name="s")
    kern = pl.kernel(
        _sc_body,
        out_type=[
            jax.ShapeDtypeStruct((_NC, _AGG_ROWS, _D), jnp.float32),
            jax.ShapeDtypeStruct((_NW, _DEG_LEN), jnp.float32),
        ],
        mesh=mesh,
        compiler_params=pltpu.CompilerParams(needs_layout_passes=False),
        scratch_types=[
            pltpu.VMEM((_CPB, _CH), jnp.int32),
            pltpu.VMEM((_CPB, _CH), jnp.int32),
            pltpu.VMEM((2, _CH, _D), jnp.float32),
            pltpu.VMEM((_DEG_LEN,), jnp.float32),
            pltpu.VMEM_SHARED((_AGG_ROWS, _D), jnp.float32),
            pltpu.SemaphoreType.DMA,
            pltpu.SemaphoreType.DMA,
        ],
    )
    return kern(m, src_p, dst_p)


def _fin_body(a0_ref, a1_ref, degp_ref, x_ref, w_ref, b_ref, o_ref):
    deg = jnp.sum(degp_ref[...], axis=1)
    agg = a0_ref[0] + a1_ref[0]
    feats = jnp.where((deg > 0.0)[:, None], agg, x_ref[...])
    o_ref[...] = _mm_t(feats, w_ref[...]) + b_ref[...]


def _finalize(agg_p, deg_p, x, w_f, b_f):
    blk = 1000
    grid = _N // blk
    return pl.pallas_call(
        _fin_body,
        grid=(grid,),
        in_specs=[
            pl.BlockSpec((1, blk, _D), lambda i: (0, i, 0)),
            pl.BlockSpec((1, blk, _D), lambda i: (1, i, 0)),
            pl.BlockSpec((blk, _NW), lambda i: (i, 0)),
            pl.BlockSpec((blk, _D), lambda i: (i, 0)),
            pl.BlockSpec((_D, _D), lambda i: (0, 0)),
            pl.BlockSpec((1, _D), lambda i: (0, 0)),
        ],
        out_specs=pl.BlockSpec((blk, _D), lambda i: (i, 0)),
        out_shape=jax.ShapeDtypeStruct((_N, _D), jnp.float32),
    )(agg_p, agg_p, deg_p, x, w_f, b_f.reshape(1, _D))


@jax.jit
def kernel(node_feats, edge_index, W_c, b_c, W_f, b_f):
    m = _compute_messages(node_feats, W_c, b_c)

    pad = _EPW_PAD - _EPW
    src = edge_index[0].reshape(_NW, _EPW)
    dst = edge_index[1].reshape(_NW, _EPW)
    src_p = jnp.pad(src, ((0, 0), (0, pad))).reshape(_NW, _NBLK, _CPB, _CH)
    dst_p = jnp.pad(dst, ((0, 0), (0, pad)),
                    constant_values=_TRASH).reshape(_NW, _NBLK, _CPB, _CH)

    agg_p, deg_p = _sc_reduce(m, src_p, dst_p)
    return _finalize(agg_p, deg_p.T[:_N], node_feats, W_f, b_f)


# speedup vs baseline: 1.0266x; 1.0266x over previous
import jax
import jax.numpy as jnp
from jax import lax
from jax.experimental import pallas as pl
from jax.experimental.pallas import tpu as pltpu
from jax.experimental.pallas import tpu_sc as plsc

_N = 10000
_E = 320000
_D = 128

_NC = 2
_NS = 16
_NW = _NC * _NS

_EPW = _E // _NW
_CH = 128
_CPB = 8
_NBLK = 10
_NCHUNK = _CPB * _NBLK
_EPW_PAD = _NCHUNK * _CH

_AGG_ROWS = 10112
_ZROWS = _AGG_ROWS // _NS
_TRASH = _N
_DEG_LEN = 10112


def _mm_t(a, b):
    return lax.dot_general(a, b, (((1,), (1,)), ((), ())),
                           preferred_element_type=jnp.float32,
                           precision=lax.Precision.HIGHEST)


def _msg_body(x_ref, w_ref, b_ref, o_ref):
    o_ref[...] = jnp.maximum(_mm_t(x_ref[...], w_ref[...]) + b_ref[...], 0.0)


def _compute_messages(x, w_c, b_c):
    blk = 1000
    grid = _N // blk
    return pl.pallas_call(
        _msg_body,
        grid=(grid,),
        in_specs=[
            pl.BlockSpec((blk, _D), lambda i: (i, 0)),
            pl.BlockSpec((_D, _D), lambda i: (0, 0)),
            pl.BlockSpec((1, _D), lambda i: (0, 0)),
        ],
        out_specs=pl.BlockSpec((blk, _D), lambda i: (i, 0)),
        out_shape=jax.ShapeDtypeStruct((_N, _D), jnp.float32),
    )(x, w_c, b_c.reshape(1, _D))


def _sc_body(m_hbm, src_hbm, dst_hbm,
             agg_out, deg_out,
             src_v, dst_v, rows_v, deg_v,
             agg_sh,
             sem0, sem1, sem_s):
    c = lax.axis_index("c")
    s = lax.axis_index("s")
    w = c * _NS + s

    pltpu.async_copy(src_hbm.at[w].at[0], src_v.at[0], sem_s)
    pltpu.async_copy(dst_hbm.at[w].at[0], dst_v.at[0], sem_s)
    pltpu.async_copy(src_hbm.at[w].at[1], src_v.at[1], sem_s)
    pltpu.async_copy(dst_hbm.at[w].at[1], dst_v.at[1], sem_s)

    def _zrow(i, _):
        rows_v[0, i // 8, pl.ds((i % 8) * 16, 16)] = jnp.zeros((16,), jnp.float32)
        return 0
    lax.fori_loop(0, _CH * 8, _zrow, 0)
    for k in range(4):
        pltpu.sync_copy(rows_v.at[0], agg_sh.at[pl.ds(s * _ZROWS + k * _CH, _CH)])
    pltpu.sync_copy(rows_v.at[0].at[pl.ds(0, _ZROWS - 4 * _CH)],
                    agg_sh.at[pl.ds(s * _ZROWS + 4 * _CH, _ZROWS - 4 * _CH)])

    def _zdeg(i, _):
        deg_v[pl.ds(i * 16, 16)] = jnp.zeros((16,), jnp.float32)
        return 0
    lax.fori_loop(0, _DEG_LEN // 16, _zdeg, 0)

    plsc.subcore_barrier()

    ones16 = jnp.ones((16,), jnp.float32)

    pltpu.make_async_copy(src_hbm.at[w].at[0], src_v.at[0], sem_s).wait()
    pltpu.make_async_copy(dst_hbm.at[w].at[0], dst_v.at[0], sem_s).wait()
    pltpu.async_copy(m_hbm.at[src_v.at[0].at[0]], rows_v.at[0], sem0)
    pltpu.async_copy(m_hbm.at[src_v.at[0].at[1]], rows_v.at[1], sem1)

    def _block(b, _):
        p = b % 2
        q = 1 - p

        def _deg_chunk(j, _):
            for r in range(_CH // 16):
                idx = dst_v[p, j, pl.ds(r * 16, 16)]
                plsc.addupdate_scatter(deg_v, [idx], ones16)
            return 0
        lax.fori_loop(0, _CPB, _deg_chunk, 0)

        for i in range(_CPB // 2 - 1):
            l0 = 2 * i
            pltpu.make_async_copy(m_hbm.at[src_v.at[p].at[l0]], rows_v.at[0], sem0).wait()
            pltpu.sync_copy(rows_v.at[0], agg_sh.at[dst_v.at[p].at[l0]], add=True)
            pltpu.async_copy(m_hbm.at[src_v.at[p].at[l0 + 2]], rows_v.at[0], sem0)

            l1 = l0 + 1
            pltpu.make_async_copy(m_hbm.at[src_v.at[p].at[l1]], rows_v.at[1], sem1).wait()
            pltpu.sync_copy(rows_v.at[1], agg_sh.at[dst_v.at[p].at[l1]], add=True)
            pltpu.async_copy(m_hbm.at[src_v.at[p].at[l1 + 2]], rows_v.at[1], sem1)

        lt = _CPB - 2
        pltpu.make_async_copy(m_hbm.at[src_v.at[p].at[lt]], rows_v.at[0], sem0).wait()
        pltpu.sync_copy(rows_v.at[0], agg_sh.at[dst_v.at[p].at[lt]], add=True)

        @pl.when(b < _NBLK - 1)
        def _():
            pltpu.make_async_copy(src_hbm.at[w].at[b + 1], src_v.at[q], sem_s).wait()
            pltpu.make_async_copy(dst_hbm.at[w].at[b + 1], dst_v.at[q], sem_s).wait()
            pltpu.async_copy(m_hbm.at[src_v.at[q].at[0]], rows_v.at[0], sem0)

        pltpu.make_async_copy(m_hbm.at[src_v.at[p].at[lt + 1]], rows_v.at[1], sem1).wait()
        pltpu.sync_copy(rows_v.at[1], agg_sh.at[dst_v.at[p].at[lt + 1]], add=True)

        @pl.when(b < _NBLK - 1)
        def _():
            pltpu.async_copy(m_hbm.at[src_v.at[q].at[1]], rows_v.at[1], sem1)

        @pl.when(b < _NBLK - 2)
        def _():
            pltpu.async_copy(src_hbm.at[w].at[b + 2], src_v.at[p], sem_s)
            pltpu.async_copy(dst_hbm.at[w].at[b + 2], dst_v.at[p], sem_s)
        return 0

    lax.fori_loop(0, _NBLK, _block, 0)

    plsc.subcore_barrier()
    pltpu.sync_copy(agg_sh.at[pl.ds(s * _ZROWS, _ZROWS)],
                    agg_out.at[c, pl.ds(s * _ZROWS, _ZROWS)])
    pltpu.sync_copy(deg_v, deg_out.at[w])


def _sc_reduce(m, src_p, dst_p):
    mesh = plsc.VectorSubcoreMesh(core_axis_name="c", subcore_axis_name="s")
    kern = pl.kernel(
        _sc_body,
        out_type=[
            jax.ShapeDtypeStruct((_NC, _AGG_ROWS, _D), jnp.float32),
            jax.ShapeDtypeStruct((_NW, _DEG_LEN), jnp.float32),
        ],
        mesh=mesh,
        compiler_params=pltpu.CompilerParams(needs_layout_passes=False),
        scratch_types=[
            pltpu.VMEM((2, _CPB, _CH), jnp.int32),
            pltpu.VMEM((2, _CPB, _CH), jnp.int32),
            pltpu.VMEM((2, _CH, _D), jnp.float32),
            pltpu.VMEM((_DEG_LEN,), jnp.float32),
            pltpu.VMEM_SHARED((_AGG_ROWS, _D), jnp.float32),
            pltpu.SemaphoreType.DMA,
            pltpu.SemaphoreType.DMA,
            pltpu.SemaphoreType.DMA,
        ],
    )
    return kern(m, src_p, dst_p)


def _fin_body(a0_ref, a1_ref, degp_ref, x_ref, w_ref, b_ref, o_ref):
    deg = jnp.sum(degp_ref[...], axis=1)
    agg = a0_ref[0] + a1_ref[0]
    feats = jnp.where((deg > 0.0)[:, None], agg, x_ref[...])
    o_ref[...] = _mm_t(feats, w_ref[...]) + b_ref[...]


def _finalize(agg_p, deg_p, x, w_f, b_f):
    blk = 1000
    grid = _N // blk
    return pl.pallas_call(
        _fin_body,
        grid=(grid,),
        in_specs=[
            pl.BlockSpec((1, blk, _D), lambda i: (0, i, 0)),
            pl.BlockSpec((1, blk, _D), lambda i: (1, i, 0)),
            pl.BlockSpec((blk, _NW), lambda i: (i, 0)),
            pl.BlockSpec((blk, _D), lambda i: (i, 0)),
            pl.BlockSpec((_D, _D), lambda i: (0, 0)),
            pl.BlockSpec((1, _D), lambda i: (0, 0)),
        ],
        out_specs=pl.BlockSpec((blk, _D), lambda i: (i, 0)),
        out_shape=jax.ShapeDtypeStruct((_N, _D), jnp.float32),
    )(agg_p, agg_p, deg_p, x, w_f, b_f.reshape(1, _D))


@jax.jit
def kernel(node_feats, edge_index, W_c, b_c, W_f, b_f):
    m = _compute_messages(node_feats, W_c, b_c)

    pad = _EPW_PAD - _EPW
    src = edge_index[0].reshape(_NW, _EPW)
    dst = edge_index[1].reshape(_NW, _EPW)
    src_p = jnp.pad(src, ((0, 0), (0, pad))).reshape(_NW, _NBLK, _CPB, _CH)
    dst_p = jnp.pad(dst, ((0, 0), (0, pad)),
                    constant_values=_TRASH).reshape(_NW, _NBLK, _CPB, _CH)

    agg_p, deg_p = _sc_reduce(m, src_p, dst_p)
    return _finalize(agg_p, deg_p.T[:_N], node_feats, W_f, b_f)
